# baseline (speedup 1.0000x reference)
import functools

import jax
import jax.numpy as jnp
from jax import lax
from jax.experimental import pallas as pl
from jax.experimental.pallas import tpu as pltpu
from jax.experimental.pallas import tpu_sc as plsc

N_NODES = 10000
N_EDGES = 320000
D_IN = 128
D_HID = 128
N_CLS = 4

NC = 2
NS = 16
NW = NC * NS
E_PER_W = N_EDGES // NW
CHUNK = 80
N_CHUNKS = E_PER_W // CHUNK
STRIPE = 624
TAIL = N_NODES - STRIPE * NS
CW = 16

_MESH = plsc.VectorSubcoreMesh(
    core_axis_name="c", subcore_axis_name="s", num_cores=NC, num_subcores=NS)


def _tile_ids():
    c = lax.axis_index("c")
    s = lax.axis_index("s")
    return c, s, c * NS + s


@functools.partial(
    pl.kernel,
    out_type=jax.ShapeDtypeStruct((NC, N_NODES, D_IN), jnp.float32),
    mesh=_MESH,
    scratch_types=[
        pltpu.VMEM((CHUNK,), jnp.int32),
        pltpu.VMEM((CHUNK,), jnp.int32),
        pltpu.VMEM((CHUNK, D_IN), jnp.float32),
        pltpu.VMEM((CHUNK,), jnp.int32),
        pltpu.VMEM((CHUNK,), jnp.int32),
        pltpu.VMEM((CHUNK, D_IN), jnp.float32),
        pltpu.VMEM_SHARED((N_NODES, D_IN), jnp.float32),
        pltpu.SemaphoreType.DMA,
        pltpu.SemaphoreType.DMA,
        pltpu.SemaphoreType.DMA,
        pltpu.SemaphoreType.DMA,
    ],
)
def _agg(feat_hbm, src_hbm, dst_hbm, zeros_hbm, sum_hbm,
         src_a, dst_a, rows_a, src_b, dst_b, rows_b, acc, sem_a, sem_b,
         ssem_a, ssem_b):
    c, s, wid = _tile_ids()
    base = pl.multiple_of(wid * E_PER_W, 8)
    stripe = pl.multiple_of(s * STRIPE, 8)
    pltpu.sync_copy(zeros_hbm.at[pl.ds(0, STRIPE), :],
                    acc.at[pl.ds(stripe, STRIPE), :])

    @pl.when(s == NS - 1)
    def _zero_tail():
        pltpu.sync_copy(zeros_hbm.at[pl.ds(0, TAIL), :],
                        acc.at[pl.ds(STRIPE * NS, TAIL), :])

    plsc.subcore_barrier()

    pltpu.sync_copy(src_hbm.at[pl.ds(base, CHUNK)], src_a)
    pltpu.sync_copy(dst_hbm.at[pl.ds(base, CHUNK)], dst_a)
    pltpu.async_copy(feat_hbm.at[src_a], rows_a, sem_a)

    def body(j, carry):
        off_b = pl.multiple_of(base + (2 * j + 1) * CHUNK, 8)
        pltpu.sync_copy(src_hbm.at[pl.ds(off_b, CHUNK)], src_b)
        pltpu.sync_copy(dst_hbm.at[pl.ds(off_b, CHUNK)], dst_b)
        pltpu.async_copy(feat_hbm.at[src_b], rows_b, sem_b)
        pltpu.make_async_copy(feat_hbm.at[src_a], rows_a, sem_a).wait()
        pltpu.async_copy(rows_a, acc.at[dst_a], add=True, sem=ssem_a)
        pltpu.make_async_copy(feat_hbm.at[src_b], rows_b, sem_b).wait()
        pltpu.async_copy(rows_b, acc.at[dst_b], add=True, sem=ssem_b)
        pltpu.make_async_copy(rows_a, acc.at[dst_a], ssem_a).wait()
        off_a = pl.multiple_of(base + (2 * j + 2) * CHUNK, 8)
        pltpu.sync_copy(src_hbm.at[pl.ds(off_a, CHUNK)], src_a)
        pltpu.sync_copy(dst_hbm.at[pl.ds(off_a, CHUNK)], dst_a)
        pltpu.async_copy(feat_hbm.at[src_a], rows_a, sem_a)
        pltpu.make_async_copy(rows_b, acc.at[dst_b], ssem_b).wait()
        return carry

    lax.fori_loop(0, (N_CHUNKS - 1) // 2, body, 0)
    pltpu.make_async_copy(feat_hbm.at[src_a], rows_a, sem_a).wait()
    pltpu.sync_copy(rows_a, acc.at[dst_a], add=True)
    plsc.subcore_barrier()
    pltpu.sync_copy(acc.at[pl.ds(stripe, STRIPE), :],
                    sum_hbm.at[c, pl.ds(stripe, STRIPE), :])

    @pl.when(s == NS - 1)
    def _copy_tail():
        pltpu.sync_copy(acc.at[pl.ds(STRIPE * NS, TAIL), :],
                        sum_hbm.at[c, pl.ds(STRIPE * NS, TAIL), :])


@functools.partial(
    pl.kernel,
    out_type=jax.ShapeDtypeStruct((NC, N_NODES, D_IN), jnp.float32),
    mesh=_MESH,
    scratch_types=[
        pltpu.VMEM((CHUNK,), jnp.int32),
        pltpu.VMEM((CHUNK,), jnp.int32),
        pltpu.VMEM((CHUNK, D_IN), jnp.float32),
        pltpu.VMEM_SHARED((N_NODES, D_IN), jnp.float32),
        pltpu.SemaphoreType.DMA,
        pltpu.SemaphoreType.DMA,
    ],
)
def _cnt(dst_hbm, zeros_hbm, ones_hbm, cnt_hbm, dst_a, dst_b, ones_v, acc,
         sem_a, sem_b):
    c, s, wid = _tile_ids()
    base = pl.multiple_of(wid * E_PER_W, 8)
    stripe = pl.multiple_of(s * STRIPE, 8)
    pltpu.sync_copy(zeros_hbm.at[pl.ds(0, STRIPE), :],
                    acc.at[pl.ds(stripe, STRIPE), :])

    @pl.when(s == NS - 1)
    def _zero_tail():
        pltpu.sync_copy(zeros_hbm.at[pl.ds(0, TAIL), :],
                        acc.at[pl.ds(STRIPE * NS, TAIL), :])

    pltpu.sync_copy(ones_hbm, ones_v)
    plsc.subcore_barrier()

    pltpu.sync_copy(dst_hbm.at[pl.ds(base, CHUNK)], dst_a)
    pltpu.async_copy(ones_v, acc.at[dst_a], add=True, sem=sem_a)

    def body(j, carry):
        off_b = pl.multiple_of(base + (2 * j + 1) * CHUNK, 8)
        pltpu.sync_copy(dst_hbm.at[pl.ds(off_b, CHUNK)], dst_b)
        pltpu.async_copy(ones_v, acc.at[dst_b], add=True, sem=sem_b)
        pltpu.make_async_copy(ones_v, acc.at[dst_a], sem_a).wait()
        off_a = pl.multiple_of(base + (2 * j + 2) * CHUNK, 8)
        pltpu.sync_copy(dst_hbm.at[pl.ds(off_a, CHUNK)], dst_a)
        pltpu.async_copy(ones_v, acc.at[dst_a], add=True, sem=sem_a)
        pltpu.make_async_copy(ones_v, acc.at[dst_b], sem_b).wait()
        return carry

    lax.fori_loop(0, (N_CHUNKS - 1) // 2, body, 0)
    pltpu.make_async_copy(ones_v, acc.at[dst_a], sem_a).wait()
    plsc.subcore_barrier()
    pltpu.sync_copy(acc.at[pl.ds(stripe, STRIPE), :],
                    cnt_hbm.at[c, pl.ds(stripe, STRIPE), :])

    @pl.when(s == NS - 1)
    def _copy_tail():
        pltpu.sync_copy(acc.at[pl.ds(STRIPE * NS, TAIL), :],
                        cnt_hbm.at[c, pl.ds(STRIPE * NS, TAIL), :])


def _dense1_body(sum_ref, cnt_ref, x_ref, w1l_ref, w1r_ref, b1_ref, w2_ref,
                 b2_ref, out_ref):
    cnt = jnp.maximum(cnt_ref[0, :, :1] + cnt_ref[1, :, :1], 1.0)
    mean = (sum_ref[0] + sum_ref[1]) / cnt
    h = jnp.maximum(
        jnp.dot(mean, w1l_ref[...], preferred_element_type=jnp.float32)
        + jnp.dot(x_ref[...], w1r_ref[...], preferred_element_type=jnp.float32)
        + b1_ref[...], 0.0)
    out_ref[...] = (
        jnp.dot(h, w2_ref[...], preferred_element_type=jnp.float32)
        + b2_ref[...])


def _dense2_body(seg_ref, cnt_ref, r2_ref, out_ref):
    cnt = jnp.maximum(cnt_ref[...], 1.0)
    out_ref[...] = (seg_ref[0] + seg_ref[1]) / cnt + r2_ref[...]


_BLK = 400


def kernel(x, edge_index, W1_l, W1_r, b1, W2_l, W2_r, b2):
    src = edge_index[0].astype(jnp.int32)
    dst = edge_index[1].astype(jnp.int32)
    zeros128 = jnp.zeros((STRIPE, D_IN), jnp.float32)
    ones128 = jnp.ones((CHUNK, D_IN), jnp.float32)

    cnt1 = _cnt(dst, zeros128, ones128)
    sum1 = _agg(x, src, dst, zeros128)

    w2 = jnp.zeros((D_HID, 128), jnp.float32)
    w2 = w2.at[:, :N_CLS].set(W2_l).at[:, CW:CW + N_CLS].set(W2_r)
    b2p = jnp.zeros((1, 128), jnp.float32).at[0, CW:CW + N_CLS].set(b2)
    proj = pl.pallas_call(
        _dense1_body,
        grid=(N_NODES // _BLK,),
        in_specs=[
            pl.BlockSpec((NC, _BLK, D_IN), lambda i: (0, i, 0)),
            pl.BlockSpec((NC, _BLK, D_IN), lambda i: (0, i, 0)),
            pl.BlockSpec((_BLK, D_IN), lambda i: (i, 0)),
            pl.BlockSpec((D_IN, D_HID), lambda i: (0, 0)),
            pl.BlockSpec((D_IN, D_HID), lambda i: (0, 0)),
            pl.BlockSpec((1, D_HID), lambda i: (0, 0)),
            pl.BlockSpec((D_HID, 128), lambda i: (0, 0)),
            pl.BlockSpec((1, 128), lambda i: (0, 0)),
        ],
        out_specs=pl.BlockSpec((_BLK, 128), lambda i: (i, 0)),
        out_shape=jax.ShapeDtypeStruct((N_NODES, 128), jnp.float32),
    )(sum1, cnt1, x, W1_l, W1_r, b1.reshape(1, D_HID), w2, b2p)

    seg2f = _agg(proj, src, dst, zeros128)

    n128 = N_NODES * CW // 128
    seg2 = seg2f[:, :, :CW].reshape(NC, n128, 128)
    cntv = cnt1[0, :, 0] + cnt1[1, :, 0]
    cnt128 = jnp.broadcast_to(cntv[:, None], (N_NODES, CW)).reshape(n128, 128)
    r2 = proj[:, CW:2 * CW].reshape(n128, 128)
    out16 = pl.pallas_call(
        _dense2_body,
        grid=(pl.cdiv(n128, _BLK),),
        in_specs=[
            pl.BlockSpec((NC, _BLK, 128), lambda i: (0, i, 0)),
            pl.BlockSpec((_BLK, 128), lambda i: (i, 0)),
            pl.BlockSpec((_BLK, 128), lambda i: (i, 0)),
        ],
        out_specs=pl.BlockSpec((_BLK, 128), lambda i: (i, 0)),
        out_shape=jax.ShapeDtypeStruct((n128, 128), jnp.float32),
    )(seg2, cnt128, r2)

    return out16.reshape(N_NODES, CW)[:, :N_CLS]


# speedup vs baseline: 1.0476x; 1.0476x over previous
import functools

import jax
import jax.numpy as jnp
from jax import lax
from jax.experimental import pallas as pl
from jax.experimental.pallas import tpu as pltpu
from jax.experimental.pallas import tpu_sc as plsc

N_NODES = 10000
N_EDGES = 320000
D_IN = 128
D_HID = 128
N_CLS = 4

NC = 2
NS = 16
NW = NC * NS
E_PER_W = N_EDGES // NW
CHUNK = 80
N_CHUNKS = E_PER_W // CHUNK
STRIPE = 624
TAIL = N_NODES - STRIPE * NS
CW = 16

_MESH = plsc.VectorSubcoreMesh(
    core_axis_name="c", subcore_axis_name="s", num_cores=NC, num_subcores=NS)


def _tile_ids():
    c = lax.axis_index("c")
    s = lax.axis_index("s")
    return c, s, c * NS + s


@functools.partial(
    pl.kernel,
    out_type=jax.ShapeDtypeStruct((NC, N_NODES, D_IN), jnp.float32),
    mesh=_MESH,
    scratch_types=[
        pltpu.VMEM((CHUNK,), jnp.int32),
        pltpu.VMEM((CHUNK,), jnp.int32),
        pltpu.VMEM((CHUNK, D_IN), jnp.float32),
        pltpu.VMEM((CHUNK,), jnp.int32),
        pltpu.VMEM((CHUNK,), jnp.int32),
        pltpu.VMEM((CHUNK, D_IN), jnp.float32),
        pltpu.VMEM_SHARED((N_NODES, D_IN), jnp.float32),
        pltpu.SemaphoreType.DMA,
        pltpu.SemaphoreType.DMA,
    ],
)
def _agg(feat_hbm, src_hbm, dst_hbm, zeros_hbm, sum_hbm,
         src_a, dst_a, rows_a, src_b, dst_b, rows_b, acc, sem_a, sem_b):
    c, s, wid = _tile_ids()
    base = pl.multiple_of(wid * E_PER_W, 8)
    stripe = pl.multiple_of(s * STRIPE, 8)
    pltpu.sync_copy(zeros_hbm.at[pl.ds(0, STRIPE), :],
                    acc.at[pl.ds(stripe, STRIPE), :])

    @pl.when(s == NS - 1)
    def _zero_tail():
        pltpu.sync_copy(zeros_hbm.at[pl.ds(0, TAIL), :],
                        acc.at[pl.ds(STRIPE * NS, TAIL), :])

    plsc.subcore_barrier()

    pltpu.sync_copy(src_hbm.at[pl.ds(base, CHUNK)], src_a)
    pltpu.sync_copy(dst_hbm.at[pl.ds(base, CHUNK)], dst_a)
    pltpu.async_copy(feat_hbm.at[src_a], rows_a, sem_a)

    def body(j, carry):
        off_b = pl.multiple_of(base + (2 * j + 1) * CHUNK, 8)
        pltpu.sync_copy(src_hbm.at[pl.ds(off_b, CHUNK)], src_b)
        pltpu.sync_copy(dst_hbm.at[pl.ds(off_b, CHUNK)], dst_b)
        pltpu.async_copy(feat_hbm.at[src_b], rows_b, sem_b)
        pltpu.make_async_copy(feat_hbm.at[src_a], rows_a, sem_a).wait()
        pltpu.sync_copy(rows_a, acc.at[dst_a], add=True)
        off_a = pl.multiple_of(base + (2 * j + 2) * CHUNK, 8)
        pltpu.sync_copy(src_hbm.at[pl.ds(off_a, CHUNK)], src_a)
        pltpu.sync_copy(dst_hbm.at[pl.ds(off_a, CHUNK)], dst_a)
        pltpu.async_copy(feat_hbm.at[src_a], rows_a, sem_a)
        pltpu.make_async_copy(feat_hbm.at[src_b], rows_b, sem_b).wait()
        pltpu.sync_copy(rows_b, acc.at[dst_b], add=True)
        return carry

    lax.fori_loop(0, (N_CHUNKS - 1) // 2, body, 0)
    pltpu.make_async_copy(feat_hbm.at[src_a], rows_a, sem_a).wait()
    pltpu.sync_copy(rows_a, acc.at[dst_a], add=True)
    plsc.subcore_barrier()
    pltpu.sync_copy(acc.at[pl.ds(stripe, STRIPE), :],
                    sum_hbm.at[c, pl.ds(stripe, STRIPE), :])

    @pl.when(s == NS - 1)
    def _copy_tail():
        pltpu.sync_copy(acc.at[pl.ds(STRIPE * NS, TAIL), :],
                        sum_hbm.at[c, pl.ds(STRIPE * NS, TAIL), :])


@functools.partial(
    pl.kernel,
    out_type=jax.ShapeDtypeStruct((NC, N_NODES, D_IN), jnp.float32),
    mesh=_MESH,
    scratch_types=[
        pltpu.VMEM((CHUNK,), jnp.int32),
        pltpu.VMEM((CHUNK,), jnp.int32),
        pltpu.VMEM((CHUNK, D_IN), jnp.float32),
        pltpu.VMEM_SHARED((N_NODES, D_IN), jnp.float32),
        pltpu.SemaphoreType.DMA,
        pltpu.SemaphoreType.DMA,
    ],
)
def _cnt(dst_hbm, zeros_hbm, ones_hbm, cnt_hbm, dst_a, dst_b, ones_v, acc,
         sem_a, sem_b):
    c, s, wid = _tile_ids()
    base = pl.multiple_of(wid * E_PER_W, 8)
    stripe = pl.multiple_of(s * STRIPE, 8)
    pltpu.sync_copy(zeros_hbm.at[pl.ds(0, STRIPE), :],
                    acc.at[pl.ds(stripe, STRIPE), :])

    @pl.when(s == NS - 1)
    def _zero_tail():
        pltpu.sync_copy(zeros_hbm.at[pl.ds(0, TAIL), :],
                        acc.at[pl.ds(STRIPE * NS, TAIL), :])

    pltpu.sync_copy(ones_hbm, ones_v)
    plsc.subcore_barrier()

    pltpu.sync_copy(dst_hbm.at[pl.ds(base, CHUNK)], dst_a)
    pltpu.async_copy(ones_v, acc.at[dst_a], add=True, sem=sem_a)

    def body(j, carry):
        off_b = pl.multiple_of(base + (2 * j + 1) * CHUNK, 8)
        pltpu.sync_copy(dst_hbm.at[pl.ds(off_b, CHUNK)], dst_b)
        pltpu.async_copy(ones_v, acc.at[dst_b], add=True, sem=sem_b)
        pltpu.make_async_copy(ones_v, acc.at[dst_a], sem_a).wait()
        off_a = pl.multiple_of(base + (2 * j + 2) * CHUNK, 8)
        pltpu.sync_copy(dst_hbm.at[pl.ds(off_a, CHUNK)], dst_a)
        pltpu.async_copy(ones_v, acc.at[dst_a], add=True, sem=sem_a)
        pltpu.make_async_copy(ones_v, acc.at[dst_b], sem_b).wait()
        return carry

    lax.fori_loop(0, (N_CHUNKS - 1) // 2, body, 0)
    pltpu.make_async_copy(ones_v, acc.at[dst_a], sem_a).wait()
    plsc.subcore_barrier()
    pltpu.sync_copy(acc.at[pl.ds(stripe, STRIPE), :],
                    cnt_hbm.at[c, pl.ds(stripe, STRIPE), :])

    @pl.when(s == NS - 1)
    def _copy_tail():
        pltpu.sync_copy(acc.at[pl.ds(STRIPE * NS, TAIL), :],
                        cnt_hbm.at[c, pl.ds(STRIPE * NS, TAIL), :])


def _dense1_body(sum_ref, cnt_ref, x_ref, w1l_ref, w1r_ref, b1_ref, w2_ref,
                 b2_ref, out_ref):
    cnt = jnp.maximum(cnt_ref[0, :, :1] + cnt_ref[1, :, :1], 1.0)
    mean = (sum_ref[0] + sum_ref[1]) / cnt
    h = jnp.maximum(
        jnp.dot(mean, w1l_ref[...], preferred_element_type=jnp.float32)
        + jnp.dot(x_ref[...], w1r_ref[...], preferred_element_type=jnp.float32)
        + b1_ref[...], 0.0)
    out_ref[...] = (
        jnp.dot(h, w2_ref[...], preferred_element_type=jnp.float32)
        + b2_ref[...])


def _dense2_body(seg_ref, cnt_ref, r2_ref, out_ref):
    cnt = jnp.maximum(cnt_ref[...], 1.0)
    out_ref[...] = (seg_ref[0] + seg_ref[1]) / cnt + r2_ref[...]


_BLK = 400


def kernel(x, edge_index, W1_l, W1_r, b1, W2_l, W2_r, b2):
    src = edge_index[0].astype(jnp.int32)
    dst = edge_index[1].astype(jnp.int32)
    zeros128 = jnp.zeros((STRIPE, D_IN), jnp.float32)
    ones128 = jnp.ones((CHUNK, D_IN), jnp.float32)

    cnt1 = _cnt(dst, zeros128, ones128)
    sum1 = _agg(x, src, dst, zeros128)

    w2 = jnp.zeros((D_HID, 128), jnp.float32)
    w2 = w2.at[:, :N_CLS].set(W2_l).at[:, CW:CW + N_CLS].set(W2_r)
    b2p = jnp.zeros((1, 128), jnp.float32).at[0, CW:CW + N_CLS].set(b2)
    proj = pl.pallas_call(
        _dense1_body,
        grid=(N_NODES // _BLK,),
        in_specs=[
            pl.BlockSpec((NC, _BLK, D_IN), lambda i: (0, i, 0)),
            pl.BlockSpec((NC, _BLK, D_IN), lambda i: (0, i, 0)),
            pl.BlockSpec((_BLK, D_IN), lambda i: (i, 0)),
            pl.BlockSpec((D_IN, D_HID), lambda i: (0, 0)),
            pl.BlockSpec((D_IN, D_HID), lambda i: (0, 0)),
            pl.BlockSpec((1, D_HID), lambda i: (0, 0)),
            pl.BlockSpec((D_HID, 128), lambda i: (0, 0)),
            pl.BlockSpec((1, 128), lambda i: (0, 0)),
        ],
        out_specs=pl.BlockSpec((_BLK, 128), lambda i: (i, 0)),
        out_shape=jax.ShapeDtypeStruct((N_NODES, 128), jnp.float32),
    )(sum1, cnt1, x, W1_l, W1_r, b1.reshape(1, D_HID), w2, b2p)

    seg2f = _agg(proj, src, dst, zeros128)

    n128 = N_NODES * CW // 128
    seg2 = seg2f[:, :, :CW].reshape(NC, n128, 128)
    cntv = cnt1[0, :, 0] + cnt1[1, :, 0]
    cnt128 = jnp.broadcast_to(cntv[:, None], (N_NODES, CW)).reshape(n128, 128)
    r2 = proj[:, CW:2 * CW].reshape(n128, 128)
    out16 = pl.pallas_call(
        _dense2_body,
        grid=(pl.cdiv(n128, _BLK),),
        in_specs=[
            pl.BlockSpec((NC, _BLK, 128), lambda i: (0, i, 0)),
            pl.BlockSpec((_BLK, 128), lambda i: (i, 0)),
            pl.BlockSpec((_BLK, 128), lambda i: (i, 0)),
        ],
        out_specs=pl.BlockSpec((_BLK, 128), lambda i: (i, 0)),
        out_shape=jax.ShapeDtypeStruct((n128, 128), jnp.float32),
    )(seg2, cnt128, r2)

    return out16.reshape(N_NODES, CW)[:, :N_CLS]
